# Initial kernel scaffold; baseline (speedup 1.0000x reference)
import jax
import jax.numpy as jnp
from jax.experimental import pallas as pl


def kernel(e, u, batch_non_omni, act_offsets, W1, b1, W2, b2, W3, b3):
    raise NotImplementedError("write your pallas kernel here")


# speedup vs baseline: 7.0809x; 7.0809x over previous
import jax
import jax.numpy as jnp
from jax.experimental import pallas as pl

N = 10000
B = 256
NP = 10240
ROWS = 1000
NEG = -1e30
IMAX = 2147483647


def _mlp_body(e_ref, w1_ref, b1_ref, w2_ref, b2_ref, w3_ref, b3_ref, out_ref):
    eb = e_ref[...]
    h = jnp.maximum(
        jnp.dot(eb, w1_ref[...], preferred_element_type=jnp.float32) + b1_ref[...], 0.0
    )
    h = jnp.maximum(
        jnp.dot(h, w2_ref[...], preferred_element_type=jnp.float32) + b2_ref[...], 0.0
    )
    out_ref[...] = (
        jnp.dot(h, w3_ref[...], preferred_element_type=jnp.float32) + b3_ref[...]
    )


def _segment_body(lg_ref, seg_ref, u_ref, off_ref, lp_ref, act_ref):
    seg = seg_ref[...]
    lg = lg_ref[...]
    ids = jax.lax.broadcasted_iota(jnp.int32, (B, NP), 0)
    mask = seg == ids

    segmax = jnp.max(jnp.where(mask, lg, NEG), axis=1, keepdims=True)
    gmax_node = jnp.sum(jnp.where(mask, segmax, 0.0), axis=0, keepdims=True)
    shifted = lg - gmax_node

    segsum = jnp.sum(jnp.where(mask, jnp.exp(shifted), 0.0), axis=1, keepdims=True)
    logz = jnp.log(segsum)
    logz_node = jnp.sum(jnp.where(mask, logz, 0.0), axis=0, keepdims=True)
    lp = shifted - logz_node
    lp_ref[...] = lp

    gum = -jnp.log(-jnp.log(u_ref[...]))
    gl = lp + gum
    gmax2 = jnp.max(jnp.where(mask, gl, NEG), axis=1, keepdims=True)
    idxs = jax.lax.broadcasted_iota(jnp.int32, (B, NP), 1)
    cand = jnp.where(mask & (gl == gmax2), idxs, jnp.int32(IMAX))
    arg = jnp.min(cand, axis=1, keepdims=True)
    act_ref[...] = arg - off_ref[...]


def kernel(e, u, batch_non_omni, act_offsets, W1, b1, W2, b2, W3, b3):
    n = e.shape[0]
    h = W1.shape[1]

    logits2 = pl.pallas_call(
        _mlp_body,
        grid=(n // ROWS,),
        in_specs=[
            pl.BlockSpec((ROWS, e.shape[1]), lambda i: (i, 0)),
            pl.BlockSpec((e.shape[1], h), lambda i: (0, 0)),
            pl.BlockSpec((1, h), lambda i: (0, 0)),
            pl.BlockSpec((h, h), lambda i: (0, 0)),
            pl.BlockSpec((1, h), lambda i: (0, 0)),
            pl.BlockSpec((h, 1), lambda i: (0, 0)),
            pl.BlockSpec((1, 1), lambda i: (0, 0)),
        ],
        out_specs=pl.BlockSpec((ROWS, 1), lambda i: (i, 0)),
        out_shape=jax.ShapeDtypeStruct((n, 1), jnp.float32),
    )(e, W1, b1.reshape(1, h), W2, b2.reshape(1, h), W3, b3.reshape(1, 1))

    logits = logits2.reshape(n)

    pad = NP - n
    lg_p = jnp.concatenate([logits, jnp.zeros((pad,), jnp.float32)]).reshape(1, NP)
    seg_p = jnp.concatenate(
        [batch_non_omni, jnp.full((pad,), B, jnp.int32)]
    ).reshape(1, NP)
    u_p = jnp.concatenate([u, jnp.full((pad,), 0.5, jnp.float32)]).reshape(1, NP)

    lp_p, act2 = pl.pallas_call(
        _segment_body,
        in_specs=[
            pl.BlockSpec((1, NP), lambda: (0, 0)),
            pl.BlockSpec((1, NP), lambda: (0, 0)),
            pl.BlockSpec((1, NP), lambda: (0, 0)),
            pl.BlockSpec((B, 1), lambda: (0, 0)),
        ],
        out_specs=[
            pl.BlockSpec((1, NP), lambda: (0, 0)),
            pl.BlockSpec((B, 1), lambda: (0, 0)),
        ],
        out_shape=[
            jax.ShapeDtypeStruct((1, NP), jnp.float32),
            jax.ShapeDtypeStruct((B, 1), jnp.int32),
        ],
    )(lg_p, seg_p, u_p, act_offsets.reshape(B, 1))

    log_probs = lp_p.reshape(NP)[:n]
    act = act2.reshape(B)
    return (logits, log_probs, act)
